# baseline (speedup 1.0000x reference)
import functools
import math

import jax
import jax.numpy as jnp
from jax import lax
from jax.experimental import pallas as pl
from jax.experimental.pallas import tpu as pltpu
from jax.experimental.pallas import tpu_sc as plsc

VOCAB = 1_000_000
EMB = 64
SCALE = math.sqrt(EMB)

NC = 2
NS = 16
NF = EMB // NC

VB_FULL = VOCAB // 128
V_TAIL = VOCAB - VB_FULL * 128
ROW_PAD = VB_FULL * 128 + 128
VB_PER_TILE = (VB_FULL + NS - 1) // NS
CH2 = 49152
N_CH2 = (VB_FULL * 128 + CH2 - 1) // CH2


def _make_detile():
    mesh = plsc.VectorSubcoreMesh(core_axis_name="c", subcore_axis_name="s")

    @functools.partial(
        pl.kernel,
        mesh=mesh,
        out_type=jax.ShapeDtypeStruct((EMB * ROW_PAD,), jnp.float32),
        scratch_types=[
            pltpu.VMEM((2, CH2), jnp.float32),
            pltpu.SemaphoreType.DMA,
            pltpu.SemaphoreType.DMA,
        ],
    )
    def detile_kernel(tab_hbm, tail_hbm, lin_hbm, buf, sem_i, sem_o):
        c = lax.axis_index("c")
        sid = lax.axis_index("s")

        def chunk_params(g):
            row_loc, c2 = lax.div(g, N_CH2), lax.rem(g, N_CH2)
            e = c * NF + sid * 2 + row_loc
            start = jnp.minimum(c2 * CH2, VB_FULL * 128 - CH2)
            return e, start

        def in_desc(g, db):
            e, start = chunk_params(g)
            return pltpu.make_async_copy(
                tab_hbm.at[e, pl.ds(start, CH2)], buf.at[db], sem_i,
            )

        def out_desc(g, db):
            e, start = chunk_params(g)
            return pltpu.make_async_copy(
                buf.at[db], lin_hbm.at[pl.ds(e * ROW_PAD + start, CH2)],
                sem_o,
            )

        NG = 2 * N_CH2
        in_desc(0, 0).start()

        def chunk_body(g, carry):
            db = lax.rem(g, 2)
            in_desc(g, db).wait()

            def scale_q(q, c3):
                buf[db, pl.ds(q * 16, 16)] = buf[db, pl.ds(q * 16, 16)] * SCALE
                return c3

            lax.fori_loop(0, CH2 // 16, scale_q, 0, unroll=8)
            out_desc(g, db).start()

            @pl.when(g >= 1)
            def _():
                out_desc(g - 1, 1 - db).wait()

            @pl.when(g + 1 < NG)
            def _():
                in_desc(g + 1, 1 - db).start()
            return carry

        lax.fori_loop(0, NG, chunk_body, 0)
        out_desc(NG - 1, lax.rem(NG - 1, 2)).wait()

        def tail_body(el, carry):
            e = c * NF + el
            pltpu.sync_copy(
                tail_hbm.at[pl.ds(e * V_TAIL, V_TAIL)],
                buf.at[0, pl.ds(0, V_TAIL)],
            )
            pltpu.sync_copy(
                buf.at[0, pl.ds(0, V_TAIL)],
                lin_hbm.at[pl.ds(e * ROW_PAD + VB_FULL * 128, V_TAIL)],
            )
            return carry

        lax.fori_loop(sid * (NF // NS), (sid + 1) * (NF // NS), tail_body, 0)

    return detile_kernel


def _make_lookup(seq, ntok):
    t_per_w = ntok // NS
    nvb = t_per_w // 128
    mesh = plsc.VectorSubcoreMesh(core_axis_name="c", subcore_axis_name="s")

    @functools.partial(
        pl.kernel,
        mesh=mesh,
        compiler_params=pltpu.CompilerParams(use_tc_tiling_on_sc=False),
        out_type=jax.ShapeDtypeStruct((seq, 8, ntok // 128, EMB // 8, 128),
                                      jnp.float32),
        scratch_types=[
            pltpu.VMEM_SHARED((VOCAB,), jnp.float32),
            pltpu.VMEM((seq, t_per_w), jnp.int32),
            pltpu.VMEM((4, t_per_w), jnp.float32),
            pltpu.SemaphoreType.DMA,
            pltpu.SemaphoreType.DMA,
            pltpu.SemaphoreType.DMA,
        ],
    )
    def lookup_kernel(tok_hbm, lin_hbm, out_hbm, feat_sh, idx_v, dst_v,
                      sem_st, sem_g, sem_o):
        c = lax.axis_index("c")
        sid = lax.axis_index("s")
        e0 = c * NF

        pltpu.sync_copy(tok_hbm.at[:, sid], idx_v)

        SC_CHUNK = 62496

        def stage(e):
            off = sid * SC_CHUNK
            pltpu.async_copy(
                lin_hbm.at[pl.ds(e * ROW_PAD + off, SC_CHUNK)],
                feat_sh.at[pl.ds(off, SC_CHUNK)],
                sem_st,
            ).wait()

            @pl.when(sid == 0)
            def _():
                pltpu.async_copy(
                    lin_hbm.at[pl.ds(e * ROW_PAD + 16 * SC_CHUNK,
                                     VOCAB - 16 * SC_CHUNK)],
                    feat_sh.at[pl.ds(16 * SC_CHUNK, VOCAB - 16 * SC_CHUNK)],
                    sem_st,
                ).wait()

        stage(e0)
        plsc.subcore_barrier()

        def feat_body(k, carry):
            e = e0 + k
            fb = lax.div(e, 8)
            fi = lax.rem(e, 8)

            def out_copy(s, db):
                return [
                    pltpu.async_copy(
                        dst_v.at[db, pl.ds(j * 128, 128)],
                        out_hbm.at[s, fb, sid * nvb + j, fi],
                        sem_o,
                    )
                    for j in range(nvb)
                ]

            NB = 4
            gcp = [None] * NB
            ocp = [None] * NB
            for s in range(seq):
                db = s % NB
                if ocp[db] is not None:
                    for cp in ocp[db]:
                        cp.wait()
                gcp[db] = pltpu.async_copy(
                    feat_sh.at[idx_v.at[s]], dst_v.at[db], sem_g
                )
                if s >= NB - 1:
                    pdb = (s - (NB - 1)) % NB
                    gcp[pdb].wait()
                    ocp[pdb] = out_copy(s - (NB - 1), pdb)
            for t in range(NB - 1):
                s = seq - (NB - 1) + t
                pdb = s % NB
                gcp[pdb].wait()
                ocp[pdb] = out_copy(s, pdb)
            for cps in ocp:
                if cps is not None:
                    for cp in cps:
                        cp.wait()

            plsc.subcore_barrier()

            @pl.when(k + 1 < NF)
            def _():
                stage(e + 1)

            plsc.subcore_barrier()
            return carry

        lax.fori_loop(0, NF, feat_body, 0)

    return lookup_kernel


@functools.lru_cache(maxsize=None)
def _kernels(seq, ntok):
    return _make_detile(), _make_lookup(seq, ntok)


@jax.jit
def kernel(tokens, table):
    ntok, seq = tokens.shape
    detile, lookup = _kernels(seq, ntok)
    tail_lin = (table[VB_FULL * 128:, :].T * SCALE).reshape(-1)
    tab_lin = detile(table.T, tail_lin)
    tok4 = tokens.T.astype(jnp.int32).reshape(seq, NS, ntok // NS)
    out5 = lookup(tok4, tab_lin)
    return out5.transpose(2, 4, 0, 1, 3).reshape(ntok, seq, EMB)


# speedup vs baseline: 1.0469x; 1.0469x over previous
import functools
import math

import jax
import jax.numpy as jnp
from jax import lax
from jax.experimental import pallas as pl
from jax.experimental.pallas import tpu as pltpu
from jax.experimental.pallas import tpu_sc as plsc

VOCAB = 1_000_000
EMB = 64
SCALE = math.sqrt(EMB)

NC = 2
NS = 16
NF = EMB // NC

VB_FULL = VOCAB // 128
V_TAIL = VOCAB - VB_FULL * 128
ROW_PAD = VB_FULL * 128 + 128
VB_PER_TILE = (VB_FULL + NS - 1) // NS
CH2 = 49152
N_CH2 = (VB_FULL * 128 + CH2 - 1) // CH2


def _make_detile():
    mesh = plsc.VectorSubcoreMesh(core_axis_name="c", subcore_axis_name="s")

    @functools.partial(
        pl.kernel,
        mesh=mesh,
        out_type=jax.ShapeDtypeStruct((EMB * ROW_PAD,), jnp.float32),
        scratch_types=[
            pltpu.VMEM((2, CH2), jnp.float32),
            pltpu.SemaphoreType.DMA,
            pltpu.SemaphoreType.DMA,
        ],
    )
    def detile_kernel(tab_hbm, tail_hbm, lin_hbm, buf, sem_i, sem_o):
        c = lax.axis_index("c")
        sid = lax.axis_index("s")

        def chunk_params(g):
            row_loc, c2 = lax.div(g, N_CH2), lax.rem(g, N_CH2)
            e = c * NF + sid * 2 + row_loc
            start = jnp.minimum(c2 * CH2, VB_FULL * 128 - CH2)
            return e, start

        def in_desc(g, db):
            e, start = chunk_params(g)
            return pltpu.make_async_copy(
                tab_hbm.at[e, pl.ds(start, CH2)], buf.at[db], sem_i,
            )

        def out_desc(g, db):
            e, start = chunk_params(g)
            return pltpu.make_async_copy(
                buf.at[db], lin_hbm.at[pl.ds(e * ROW_PAD + start, CH2)],
                sem_o,
            )

        NG = 2 * N_CH2
        in_desc(0, 0).start()

        def chunk_body(g, carry):
            db = lax.rem(g, 2)
            in_desc(g, db).wait()

            def scale_q(q, c3):
                buf[db, pl.ds(q * 16, 16)] = buf[db, pl.ds(q * 16, 16)] * SCALE
                return c3

            lax.fori_loop(0, CH2 // 16, scale_q, 0, unroll=8)
            out_desc(g, db).start()

            @pl.when(g >= 1)
            def _():
                out_desc(g - 1, 1 - db).wait()

            @pl.when(g + 1 < NG)
            def _():
                in_desc(g + 1, 1 - db).start()
            return carry

        lax.fori_loop(0, NG, chunk_body, 0)
        out_desc(NG - 1, lax.rem(NG - 1, 2)).wait()

        def tail_body(el, carry):
            e = c * NF + el
            pltpu.sync_copy(
                tail_hbm.at[pl.ds(e * V_TAIL, V_TAIL)],
                buf.at[0, pl.ds(0, V_TAIL)],
            )
            pltpu.sync_copy(
                buf.at[0, pl.ds(0, V_TAIL)],
                lin_hbm.at[pl.ds(e * ROW_PAD + VB_FULL * 128, V_TAIL)],
            )
            return carry

        lax.fori_loop(sid * (NF // NS), (sid + 1) * (NF // NS), tail_body, 0)

    return detile_kernel


def _make_lookup(seq, ntok):
    t_per_w = ntok // NS
    nvb = t_per_w // 128
    mesh = plsc.VectorSubcoreMesh(core_axis_name="c", subcore_axis_name="s")

    @functools.partial(
        pl.kernel,
        mesh=mesh,
        compiler_params=pltpu.CompilerParams(use_tc_tiling_on_sc=False),
        out_type=jax.ShapeDtypeStruct((seq, 8, ntok // 128, EMB // 8, 128),
                                      jnp.float32),
        scratch_types=[
            pltpu.VMEM_SHARED((VOCAB,), jnp.float32),
            pltpu.VMEM((seq // 2, 2 * t_per_w), jnp.int32),
            pltpu.VMEM((4, 2 * t_per_w), jnp.float32),
            pltpu.SemaphoreType.DMA,
            pltpu.SemaphoreType.DMA,
            pltpu.SemaphoreType.DMA,
        ],
    )
    def lookup_kernel(tok_hbm, lin_hbm, out_hbm, feat_sh, idx_v, dst_v,
                      sem_st, sem_g, sem_o):
        c = lax.axis_index("c")
        sid = lax.axis_index("s")
        e0 = c * NF

        pltpu.sync_copy(tok_hbm.at[:, sid], idx_v)

        def stage(e):
            @pl.when(sid == 0)
            def _():
                pltpu.async_copy(
                    lin_hbm.at[pl.ds(e * ROW_PAD, VOCAB)], feat_sh, sem_st
                ).wait()

        stage(e0)
        plsc.subcore_barrier()

        def feat_body(k, carry):
            e = e0 + k
            fb = lax.div(e, 8)
            fi = lax.rem(e, 8)

            def out_copy(sp, db):
                return [
                    pltpu.async_copy(
                        dst_v.at[db, pl.ds(j * 128, 128)],
                        out_hbm.at[2 * sp + j // nvb, fb,
                                   sid * nvb + j % nvb, fi],
                        sem_o,
                    )
                    for j in range(2 * nvb)
                ]

            NB = 4
            NP = seq // 2
            gcp = [None] * NB
            ocp = [None] * NB
            for sp in range(NP):
                db = sp % NB
                if ocp[db] is not None:
                    for cp in ocp[db]:
                        cp.wait()
                gcp[db] = pltpu.async_copy(
                    feat_sh.at[idx_v.at[sp]], dst_v.at[db], sem_g
                )
                if sp >= NB - 1:
                    pdb = (sp - (NB - 1)) % NB
                    gcp[pdb].wait()
                    ocp[pdb] = out_copy(sp - (NB - 1), pdb)
            for t in range(NB - 1):
                sp = NP - (NB - 1) + t
                pdb = sp % NB
                gcp[pdb].wait()
                ocp[pdb] = out_copy(sp, pdb)
            for cps in ocp:
                if cps is not None:
                    for cp in cps:
                        cp.wait()

            plsc.subcore_barrier()

            @pl.when(k + 1 < NF)
            def _():
                stage(e + 1)

            plsc.subcore_barrier()
            return carry

        lax.fori_loop(0, NF, feat_body, 0)

    return lookup_kernel


@functools.lru_cache(maxsize=None)
def _kernels(seq, ntok):
    return _make_detile(), _make_lookup(seq, ntok)


@jax.jit
def kernel(tokens, table):
    ntok, seq = tokens.shape
    detile, lookup = _kernels(seq, ntok)
    tail_lin = (table[VB_FULL * 128:, :].T * SCALE).reshape(-1)
    tab_lin = detile(table.T, tail_lin)
    tpw = ntok // NS
    tok4 = (tokens.T.astype(jnp.int32)
            .reshape(seq // 2, 2, NS, tpw)
            .transpose(0, 2, 1, 3)
            .reshape(seq // 2, NS, 2 * tpw))
    out5 = lookup(tok4, tab_lin)
    return out5.transpose(2, 4, 0, 1, 3).reshape(ntok, seq, EMB)


# speedup vs baseline: 1.0469x; 1.0000x over previous
import functools
import math

import jax
import jax.numpy as jnp
from jax import lax
from jax.experimental import pallas as pl
from jax.experimental.pallas import tpu as pltpu
from jax.experimental.pallas import tpu_sc as plsc

VOCAB = 1_000_000
EMB = 64
SCALE = math.sqrt(EMB)

NC = 2
NS = 16
NF = EMB // NC

VB_FULL = VOCAB // 128
V_TAIL = VOCAB - VB_FULL * 128
ROW_PAD = VB_FULL * 128 + 128
VB_PER_TILE = (VB_FULL + NS - 1) // NS
CH2 = 49152
N_CH2 = (VB_FULL * 128 + CH2 - 1) // CH2


def _make_detile():
    mesh = plsc.VectorSubcoreMesh(core_axis_name="c", subcore_axis_name="s")

    @functools.partial(
        pl.kernel,
        mesh=mesh,
        out_type=jax.ShapeDtypeStruct((EMB * ROW_PAD,), jnp.float32),
        scratch_types=[
            pltpu.VMEM((2, CH2), jnp.float32),
            pltpu.SemaphoreType.DMA,
            pltpu.SemaphoreType.DMA,
        ],
    )
    def detile_kernel(tab_hbm, tail_hbm, lin_hbm, buf, sem_i, sem_o):
        c = lax.axis_index("c")
        sid = lax.axis_index("s")

        def chunk_params(g):
            row_loc, c2 = lax.div(g, N_CH2), lax.rem(g, N_CH2)
            e = c * NF + sid * 2 + row_loc
            start = jnp.minimum(c2 * CH2, VB_FULL * 128 - CH2)
            return e, start

        def in_desc(g, db):
            e, start = chunk_params(g)
            return pltpu.make_async_copy(
                tab_hbm.at[e, pl.ds(start, CH2)], buf.at[db], sem_i,
            )

        def out_desc(g, db):
            e, start = chunk_params(g)
            return pltpu.make_async_copy(
                buf.at[db], lin_hbm.at[pl.ds(e * ROW_PAD + start, CH2)],
                sem_o,
            )

        NG = 2 * N_CH2
        in_desc(0, 0).start()

        def chunk_body(g, carry):
            db = lax.rem(g, 2)
            in_desc(g, db).wait()

            def scale_q(q, c3):
                buf[db, pl.ds(q * 16, 16)] = buf[db, pl.ds(q * 16, 16)] * SCALE
                return c3

            lax.fori_loop(0, CH2 // 16, scale_q, 0, unroll=8)
            out_desc(g, db).start()

            @pl.when(g >= 1)
            def _():
                out_desc(g - 1, 1 - db).wait()

            @pl.when(g + 1 < NG)
            def _():
                in_desc(g + 1, 1 - db).start()
            return carry

        lax.fori_loop(0, NG, chunk_body, 0)
        out_desc(NG - 1, lax.rem(NG - 1, 2)).wait()

        def tail_body(el, carry):
            e = c * NF + el
            pltpu.sync_copy(
                tail_hbm.at[pl.ds(e * V_TAIL, V_TAIL)],
                buf.at[0, pl.ds(0, V_TAIL)],
            )
            pltpu.sync_copy(
                buf.at[0, pl.ds(0, V_TAIL)],
                lin_hbm.at[pl.ds(e * ROW_PAD + VB_FULL * 128, V_TAIL)],
            )
            return carry

        lax.fori_loop(sid * (NF // NS), (sid + 1) * (NF // NS), tail_body, 0)

    return detile_kernel


def _make_lookup(seq, ntok):
    t_per_w = ntok // NS
    nvb = t_per_w // 128
    mesh = plsc.VectorSubcoreMesh(core_axis_name="c", subcore_axis_name="s")

    @functools.partial(
        pl.kernel,
        mesh=mesh,
        compiler_params=pltpu.CompilerParams(use_tc_tiling_on_sc=False),
        out_type=jax.ShapeDtypeStruct((seq, 8, ntok // 128, EMB // 8, 128),
                                      jnp.float32),
        scratch_types=[
            pltpu.VMEM_SHARED((VOCAB,), jnp.float32),
            pltpu.VMEM((seq // 2, 2 * t_per_w), jnp.int32),
            pltpu.VMEM((4, 2 * t_per_w), jnp.float32),
            pltpu.SemaphoreType.DMA,
            pltpu.SemaphoreType.DMA,
            pltpu.SemaphoreType.DMA,
        ],
    )
    def lookup_kernel(tok_hbm, lin_hbm, out_hbm, feat_sh, idx_v, dst_v,
                      sem_st, sem_g, sem_o):
        c = lax.axis_index("c")
        sid = lax.axis_index("s")
        e0 = c * NF

        pltpu.sync_copy(tok_hbm.at[:, sid], idx_v)

        HALF = VOCAB // 2

        def stage(e):
            @pl.when(sid == 0)
            def _():
                pltpu.async_copy(
                    lin_hbm.at[pl.ds(e * ROW_PAD, HALF)],
                    feat_sh.at[pl.ds(0, HALF)], sem_st,
                ).wait()

            @pl.when(sid == 8)
            def _():
                pltpu.async_copy(
                    lin_hbm.at[pl.ds(e * ROW_PAD + HALF, VOCAB - HALF)],
                    feat_sh.at[pl.ds(HALF, VOCAB - HALF)], sem_st,
                ).wait()

        stage(e0)
        plsc.subcore_barrier()

        def feat_body(k, carry):
            e = e0 + k
            fb = lax.div(e, 8)
            fi = lax.rem(e, 8)

            def out_copy(sp, db):
                return [
                    pltpu.async_copy(
                        dst_v.at[db, pl.ds(j * 128, 128)],
                        out_hbm.at[2 * sp + j // nvb, fb,
                                   sid * nvb + j % nvb, fi],
                        sem_o,
                    )
                    for j in range(2 * nvb)
                ]

            NB = 4
            NP = seq // 2
            gcp = [None] * NB
            ocp = [None] * NB
            for sp in range(NP):
                db = sp % NB
                if ocp[db] is not None:
                    for cp in ocp[db]:
                        cp.wait()
                gcp[db] = pltpu.async_copy(
                    feat_sh.at[idx_v.at[sp]], dst_v.at[db], sem_g
                )
                if sp >= NB - 1:
                    pdb = (sp - (NB - 1)) % NB
                    gcp[pdb].wait()
                    ocp[pdb] = out_copy(sp - (NB - 1), pdb)
            for t in range(NB - 1):
                sp = NP - (NB - 1) + t
                pdb = sp % NB
                gcp[pdb].wait()
                ocp[pdb] = out_copy(sp, pdb)
            plsc.subcore_barrier()

            @pl.when(k + 1 < NF)
            def _():
                stage(e + 1)

            for cps in ocp:
                if cps is not None:
                    for cp in cps:
                        cp.wait()

            plsc.subcore_barrier()
            return carry

        lax.fori_loop(0, NF, feat_body, 0)

    return lookup_kernel


@functools.lru_cache(maxsize=None)
def _kernels(seq, ntok):
    return _make_detile(), _make_lookup(seq, ntok)


@jax.jit
def kernel(tokens, table):
    ntok, seq = tokens.shape
    detile, lookup = _kernels(seq, ntok)
    tail_lin = (table[VB_FULL * 128:, :].T * SCALE).reshape(-1)
    tab_lin = detile(table.T, tail_lin)
    tpw = ntok // NS
    tok4 = (tokens.T.astype(jnp.int32)
            .reshape(seq // 2, 2, NS, tpw)
            .transpose(0, 2, 1, 3)
            .reshape(seq // 2, NS, 2 * tpw))
    out5 = lookup(tok4, tab_lin)
    return out5.transpose(2, 4, 0, 1, 3).reshape(ntok, seq, EMB)
